# baseline (speedup 1.0000x reference)
import jax
import jax.numpy as jnp
from jax.experimental import pallas as pl


def _body(x_ref, mask_ref, out_ref):
    x = x_ref[...]
    m = jnp.max(x, axis=1)
    msk = mask_ref[0, 0]
    val = jnp.sum(m * msk) / jnp.sum(msk)
    out_ref[...] = jnp.full(out_ref.shape, val, out_ref.dtype)


def kernel(ptm, mem_mask):
    nframes, nseq, nmem, c, w, h = ptm.shape
    ns = nframes * nseq
    cwh = c * w * h
    x = ptm.reshape(ns * nmem, cwh)
    maskf = mem_mask.reshape(ns, 1, nmem).astype(ptm.dtype)
    out = pl.pallas_call(
        _body,
        grid=(ns,),
        in_specs=[
            pl.BlockSpec((nmem, cwh), lambda i: (i, 0)),
            pl.BlockSpec((1, 1, nmem), lambda i: (i, 0, 0)),
        ],
        out_specs=pl.BlockSpec((1, 1, 128), lambda i: (i, 0, 0)),
        out_shape=jax.ShapeDtypeStruct((ns, 1, 128), ptm.dtype),
    )(x, maskf)
    return out[:, 0, 0]


# speedup vs baseline: 1.3741x; 1.3741x over previous
import jax
import jax.numpy as jnp
from jax.experimental import pallas as pl


def _body(x_ref, mask_ref, out_ref):
    x = x_ref[...]
    m = jnp.max(x, axis=(1, 2))
    msk = mask_ref[0, 0]
    val = jnp.sum(m * msk) / jnp.sum(msk)
    out_ref[...] = jnp.full(out_ref.shape, val, out_ref.dtype)


def kernel(ptm, mem_mask):
    nframes, nseq, nmem, c, w, h = ptm.shape
    ns = nframes * nseq
    cwh = c * w * h
    x = ptm.reshape(ns * nmem, cwh // 128, 128)
    maskf = mem_mask.reshape(ns, 1, nmem).astype(ptm.dtype)
    out = pl.pallas_call(
        _body,
        grid=(ns,),
        in_specs=[
            pl.BlockSpec((nmem, cwh // 128, 128), lambda i: (i, 0, 0)),
            pl.BlockSpec((1, 1, nmem), lambda i: (i, 0, 0)),
        ],
        out_specs=pl.BlockSpec((1, 1, 128), lambda i: (i, 0, 0)),
        out_shape=jax.ShapeDtypeStruct((ns, 1, 128), ptm.dtype),
    )(x, maskf)
    return out[:, 0, 0]


# speedup vs baseline: 8.4469x; 6.1471x over previous
import jax
import jax.numpy as jnp
from jax.experimental import pallas as pl


def _body(x_ref, mask_ref, out_ref):
    x = x_ref[...]
    m = jnp.max(x, axis=(1, 2))
    msk = mask_ref[0, 0]
    val = jnp.sum(m * msk) / jnp.sum(msk)
    out_ref[...] = jnp.full(out_ref.shape, val, out_ref.dtype)


def kernel(ptm, mem_mask):
    nframes, nseq, nmem, c, w, h = ptm.shape
    ns = nframes * nseq
    x = ptm.transpose(0, 1, 2, 4, 5, 3).reshape(ns * nmem, w * h, c)
    maskf = mem_mask.reshape(ns, 1, nmem).astype(ptm.dtype)
    out = pl.pallas_call(
        _body,
        grid=(ns,),
        in_specs=[
            pl.BlockSpec((nmem, w * h, c), lambda i: (i, 0, 0)),
            pl.BlockSpec((1, 1, nmem), lambda i: (i, 0, 0)),
        ],
        out_specs=pl.BlockSpec((1, 1, 128), lambda i: (i, 0, 0)),
        out_shape=jax.ShapeDtypeStruct((ns, 1, 128), ptm.dtype),
    )(x, maskf)
    return out[:, 0, 0]
